# baseline (speedup 1.0000x reference)
import functools

import jax
import jax.numpy as jnp
from jax import lax
from jax.experimental import pallas as pl
from jax.experimental.pallas import tpu as pltpu
from jax.experimental.pallas import tpu_sc as plsc

_B = 16384
_V = 100000
_NC = 2
_NS = 16
_NW = _NC * _NS
_HALF = _B // 2
_EPS = 1e-8


def _gather_body(u0i, u1i, u2i, i0i, i1i, i2i,
                 U0t, U1t, U2t, I0t, I1t, I2t,
                 out, trow, idxv, outv, osem):
    wid = lax.axis_index("s") * _NC + lax.axis_index("c")
    sub = wid - 16

    def stage(tab, row, idx_hbm):
        ct = pltpu.async_copy(tab.at[row], trow, osem)
        ci = pltpu.async_copy(idx_hbm, idxv, osem)
        ct.wait()
        ci.wait()

    def gather_to(drow):
        for half in range(2):
            @pl.loop(0, _HALF // 16, unroll=2)
            def _(i):
                v = idxv[pl.ds(half * _HALF + i * 16, 16)]
                outv[pl.ds(i * 16, 16)] = plsc.load_gather(trow, [v])
            pltpu.sync_copy(outv, out.at[drow, pl.ds(half * _HALF, _HALF)])

    stage(U0t, wid, u0i)
    gather_to(wid)
    @pl.when(wid < 16)
    def _():
        stage(U1t, wid, u1i)
    @pl.when(wid >= 16)
    def _():
        stage(U2t, sub, u2i)
    gather_to(32 + wid)
    stage(I0t, wid, i0i)
    gather_to(64 + wid)
    @pl.when(wid < 16)
    def _():
        stage(I1t, wid, i1i)
    @pl.when(wid >= 16)
    def _():
        stage(I2t, sub, i2i)
    gather_to(96 + wid)


@functools.cache
def _make_gather():
    return functools.partial(
        pl.kernel,
        out_type=jax.ShapeDtypeStruct((128, _B), jnp.float32),
        mesh=plsc.VectorSubcoreMesh(core_axis_name="c", subcore_axis_name="s"),
        compiler_params=pltpu.CompilerParams(
            use_tc_tiling_on_sc=True, needs_layout_passes=False),
        scratch_types=[
            pltpu.VMEM((_V,), jnp.float32),
            pltpu.VMEM((_B,), jnp.int32),
            pltpu.VMEM((_HALF,), jnp.float32),
            pltpu.SemaphoreType.DMA,
        ],
    )(_gather_body)


def _mlp_body(xT, W1t, b1, W2t, b2, W3t, b3, Wot, bo, out):
    x = xT[...]
    a = x[33:64, :]
    s = jnp.sum(a * a, axis=0, keepdims=True)
    na = jnp.sqrt(s)
    d = jnp.maximum(na, _EPS)
    cos = s / (d * d)
    h = jnp.maximum(jnp.dot(W1t[...], x, preferred_element_type=jnp.float32) + b1[...], 0.0)
    h = jnp.maximum(jnp.dot(W2t[...], h, preferred_element_type=jnp.float32) + b2[...], 0.0)
    h = jnp.maximum(jnp.dot(W3t[...], h, preferred_element_type=jnp.float32) + b3[...], 0.0)
    hc = jnp.concatenate([h, cos], axis=0)
    out[...] = jnp.dot(Wot[...], hc, preferred_element_type=jnp.float32) + bo[...]


def _mlp_t(xT, W1t, b1, W2t, b2, W3t, b3, Wot, bo):
    blk = 8192
    grid = (_B // blk,)
    col = lambda h: pl.BlockSpec((h, blk), lambda i: (0, i))
    rep = lambda a, b: pl.BlockSpec((a, b), lambda i: (0, 0))
    return pl.pallas_call(
        _mlp_body,
        grid=grid,
        in_specs=[
            col(128),
            rep(64, 128), rep(64, 1), rep(32, 64), rep(32, 1),
            rep(16, 32), rep(16, 1), rep(1, 17), rep(1, 1),
        ],
        out_specs=pl.BlockSpec((1, blk), lambda i: (0, i)),
        out_shape=jax.ShapeDtypeStruct((1, _B), jnp.float32),
    )(xT, W1t, b1, W2t, b2, W3t, b3, Wot, bo)


def kernel(user_idx, item_idx, U0, U1, U2, I0, I1, I2,
           W1, b1, W2, b2, W3, b3, Wo, bo):
    ui = user_idx.astype(jnp.int32)
    ii = item_idx.astype(jnp.int32)
    xT = _make_gather()(
        ui[:, 0], ui[:, 1], ui[:, 2], ii[:, 0], ii[:, 1], ii[:, 2],
        U0.T, U1.T, U2.T, I0.T, I1.T, I2.T)
    outT = _mlp_t(xT, W1.T, b1.reshape(-1, 1), W2.T, b2.reshape(-1, 1),
                  W3.T, b3.reshape(-1, 1), Wo.T, bo.reshape(1, 1))
    return outT.reshape(_B, 1)


# speedup vs baseline: 1.0681x; 1.0681x over previous
import functools

import jax
import jax.numpy as jnp
from jax import lax
from jax.experimental import pallas as pl
from jax.experimental.pallas import tpu as pltpu
from jax.experimental.pallas import tpu_sc as plsc

_B = 16384
_V = 100000
_NC = 2
_NS = 16
_NW = _NC * _NS
_HALF = _B // 2
_EPS = 1e-8


def _gather_body(u0i, u1i, u2i, i0i, i1i, i2i,
                 U0t, U1t, U2t, I0t, I1t, I2t,
                 out, trow, idxv, outv, osem):
    wid = lax.axis_index("s") * _NC + lax.axis_index("c")
    sub = wid - 16

    def stage(tab, row, idx_hbm):
        ct = pltpu.async_copy(tab.at[row], trow, osem)
        ci = pltpu.async_copy(idx_hbm, idxv, osem)
        ct.wait()
        ci.wait()

    def gather_to(drow):
        for half in range(2):
            @pl.loop(0, _HALF // 16)
            def _(i):
                v = idxv[pl.ds(half * _HALF + i * 16, 16)]
                outv[pl.ds(i * 16, 16)] = plsc.load_gather(trow, [v])
            pltpu.sync_copy(outv, out.at[drow, pl.ds(half * _HALF, _HALF)])

    stage(U0t, wid, u0i)
    gather_to(wid)
    @pl.when(wid < 16)
    def _():
        stage(U1t, wid, u1i)
    @pl.when(wid >= 16)
    def _():
        stage(U2t, sub, u2i)
    gather_to(32 + wid)
    stage(I0t, wid, i0i)
    gather_to(64 + wid)
    @pl.when(wid < 16)
    def _():
        stage(I1t, wid, i1i)
    @pl.when(wid >= 16)
    def _():
        stage(I2t, sub, i2i)
    gather_to(96 + wid)


@functools.cache
def _make_gather():
    return functools.partial(
        pl.kernel,
        out_type=jax.ShapeDtypeStruct((128, _B), jnp.float32),
        mesh=plsc.VectorSubcoreMesh(core_axis_name="c", subcore_axis_name="s"),
        compiler_params=pltpu.CompilerParams(
            use_tc_tiling_on_sc=True, needs_layout_passes=False),
        scratch_types=[
            pltpu.VMEM((_V,), jnp.float32),
            pltpu.VMEM((_B,), jnp.int32),
            pltpu.VMEM((_HALF,), jnp.float32),
            pltpu.SemaphoreType.DMA,
        ],
    )(_gather_body)


def _mlp_body(xT, W1t, b1, W2t, b2, W3t, b3, Wot, bo, out):
    x = xT[...]
    a = x[33:64, :]
    s = jnp.sum(a * a, axis=0, keepdims=True)
    na = jnp.sqrt(s)
    d = jnp.maximum(na, _EPS)
    cos = s / (d * d)
    h = jnp.maximum(jnp.dot(W1t[...], x, preferred_element_type=jnp.float32) + b1[...], 0.0)
    h = jnp.maximum(jnp.dot(W2t[...], h, preferred_element_type=jnp.float32) + b2[...], 0.0)
    h = jnp.maximum(jnp.dot(W3t[...], h, preferred_element_type=jnp.float32) + b3[...], 0.0)
    hc = jnp.concatenate([h, cos], axis=0)
    out[...] = jnp.dot(Wot[...], hc, preferred_element_type=jnp.float32) + bo[...]


def _mlp_t(xT, W1t, b1, W2t, b2, W3t, b3, Wot, bo):
    blk = 8192
    grid = (_B // blk,)
    col = lambda h: pl.BlockSpec((h, blk), lambda i: (0, i))
    rep = lambda a, b: pl.BlockSpec((a, b), lambda i: (0, 0))
    return pl.pallas_call(
        _mlp_body,
        grid=grid,
        in_specs=[
            col(128),
            rep(64, 128), rep(64, 1), rep(32, 64), rep(32, 1),
            rep(16, 32), rep(16, 1), rep(1, 17), rep(1, 1),
        ],
        out_specs=pl.BlockSpec((1, blk), lambda i: (0, i)),
        out_shape=jax.ShapeDtypeStruct((1, _B), jnp.float32),
    )(xT, W1t, b1, W2t, b2, W3t, b3, Wot, bo)


def kernel(user_idx, item_idx, U0, U1, U2, I0, I1, I2,
           W1, b1, W2, b2, W3, b3, Wo, bo):
    ui = user_idx.astype(jnp.int32)
    ii = item_idx.astype(jnp.int32)
    xT = _make_gather()(
        ui[:, 0], ui[:, 1], ui[:, 2], ii[:, 0], ii[:, 1], ii[:, 2],
        U0.T, U1.T, U2.T, I0.T, I1.T, I2.T)
    outT = _mlp_t(xT, W1.T, b1.reshape(-1, 1), W2.T, b2.reshape(-1, 1),
                  W3.T, b3.reshape(-1, 1), Wo.T, bo.reshape(1, 1))
    return outT.reshape(_B, 1)
